# baseline (speedup 1.0000x reference)
import jax
import jax.numpy as jnp
from jax.experimental import pallas as pl
from jax.experimental.pallas import tpu as pltpu

_TM = 512
_WSTRIPES = 4
_S = 4
_SM = _TM // _S
_NXB = 3
_NOB = 3


def _linear_kernel(x_hbm, w_hbm, b_ref, o_hbm,
                   w_vmem, x_buf, o_buf, in_sem, out_sem, w_sem):
    n_steps = x_hbm.shape[0] // _TM
    wk = w_hbm.shape[0] // _WSTRIPES

    def w_stripe(q):
        return pltpu.make_async_copy(
            w_hbm.at[pl.ds(q * wk, wk)], w_vmem.at[pl.ds(q * wk, wk)],
            w_sem.at[q],
        )

    def dma_in(slot, step):
        for h in range(_S):
            pltpu.make_async_copy(
                x_hbm.at[pl.ds(step * _TM + h * _SM, _SM)],
                x_buf.at[slot].at[pl.ds(h * _SM, _SM)],
                in_sem.at[slot, h],
            ).start()

    def wait_in(slot):
        for h in range(_S):
            pltpu.make_async_copy(
                x_hbm.at[pl.ds(0, _SM)],
                x_buf.at[slot].at[pl.ds(0, _SM)],
                in_sem.at[slot, h],
            ).wait()

    def dma_out(slot, step):
        for h in range(_S):
            pltpu.make_async_copy(
                o_buf.at[slot].at[pl.ds(h * _SM, _SM)],
                o_hbm.at[pl.ds(step * _TM + h * _SM, _SM)],
                out_sem.at[slot, h],
            ).start()

    def wait_out(slot):
        for h in range(_S):
            pltpu.make_async_copy(
                o_buf.at[slot].at[pl.ds(0, _SM)],
                o_hbm.at[pl.ds(0, _SM)],
                out_sem.at[slot, h],
            ).wait()

    w_stripe(0).start()
    dma_in(0, 0)
    for q in range(1, _WSTRIPES):
        w_stripe(q).start()
    dma_in(1, 1)
    dma_in(2, 2)

    w_stripe(0).wait()
    wait_in(0)
    x0 = x_buf[0]
    o_buf[0] = b_ref[...] + jnp.dot(
        x0[:, 0:wk], w_vmem[0:wk, :], preferred_element_type=jnp.float32
    )
    for q in range(1, _WSTRIPES):
        w_stripe(q).wait()
        o_buf[0] += jnp.dot(
            x0[:, q * wk:(q + 1) * wk], w_vmem[q * wk:(q + 1) * wk, :],
            preferred_element_type=jnp.float32,
        )
    dma_out(0, 0)

    def body(step, _):
        cur = jax.lax.rem(step, _NXB)
        pre = jax.lax.rem(step + 2, _NXB)
        ocur = jax.lax.rem(step, _NOB)

        @pl.when(step + 2 < n_steps)
        def _prefetch():
            dma_in(pre, step + 2)

        wait_in(cur)

        @pl.when(step >= _NOB)
        def _drain():
            wait_out(ocur)

        o_buf[ocur] = (
            jnp.dot(x_buf[cur], w_vmem[...], preferred_element_type=jnp.float32)
            + b_ref[...]
        )
        dma_out(ocur, step)
        return ()

    jax.lax.fori_loop(1, n_steps, body, (), unroll=4)
    for t in range(n_steps - _NOB, n_steps):
        wait_out(jax.lax.rem(t, _NOB))


def kernel(x, w, b):
    m, k = x.shape
    n = w.shape[1]
    cost = pl.CostEstimate(
        flops=2 * m * k * n,
        transcendentals=0,
        bytes_accessed=4 * m * k + 4 * k * n + 4 * n + 4 * m * n,
    )
    return pl.pallas_call(
        _linear_kernel,
        out_shape=jax.ShapeDtypeStruct((m, n), jnp.float32),
        in_specs=[
            pl.BlockSpec(memory_space=pltpu.MemorySpace.HBM),
            pl.BlockSpec(memory_space=pltpu.MemorySpace.HBM),
            pl.BlockSpec(memory_space=pltpu.MemorySpace.VMEM),
        ],
        out_specs=pl.BlockSpec(memory_space=pltpu.MemorySpace.HBM),
        scratch_shapes=[
            pltpu.VMEM((k, n), jnp.float32),
            pltpu.VMEM((_NXB, _TM, k), jnp.float32),
            pltpu.VMEM((_NOB, _TM, n), jnp.float32),
            pltpu.SemaphoreType.DMA((_NXB, _S)),
            pltpu.SemaphoreType.DMA((_NOB, _S)),
            pltpu.SemaphoreType.DMA((_WSTRIPES,)),
        ],
        compiler_params=pltpu.CompilerParams(
            vmem_limit_bytes=60 << 20,
        ),
        cost_estimate=cost,
    )(x, w, b)


# speedup vs baseline: 1.0009x; 1.0009x over previous
import jax
import jax.numpy as jnp
from jax.experimental import pallas as pl
from jax.experimental.pallas import tpu as pltpu

_TM = 512
_WSTRIPES = 4
_S = 2
_SM = _TM // _S
_NXB = 3
_NOB = 3


def _linear_kernel(x_hbm, w_hbm, b_ref, o_hbm,
                   w_vmem, x_buf, o_buf, in_sem, out_sem, w_sem):
    n_steps = x_hbm.shape[0] // _TM
    wk = w_hbm.shape[0] // _WSTRIPES

    def w_stripe(q):
        return pltpu.make_async_copy(
            w_hbm.at[pl.ds(q * wk, wk)], w_vmem.at[pl.ds(q * wk, wk)],
            w_sem.at[q],
        )

    def dma_in(slot, step):
        for h in range(_S):
            pltpu.make_async_copy(
                x_hbm.at[pl.ds(step * _TM + h * _SM, _SM)],
                x_buf.at[slot].at[pl.ds(h * _SM, _SM)],
                in_sem.at[slot, h],
            ).start()

    def wait_in(slot):
        for h in range(_S):
            pltpu.make_async_copy(
                x_hbm.at[pl.ds(0, _SM)],
                x_buf.at[slot].at[pl.ds(0, _SM)],
                in_sem.at[slot, h],
            ).wait()

    def dma_out(slot, step):
        for h in range(_S):
            pltpu.make_async_copy(
                o_buf.at[slot].at[pl.ds(h * _SM, _SM)],
                o_hbm.at[pl.ds(step * _TM + h * _SM, _SM)],
                out_sem.at[slot, h],
            ).start()

    def wait_out(slot):
        for h in range(_S):
            pltpu.make_async_copy(
                o_buf.at[slot].at[pl.ds(0, _SM)],
                o_hbm.at[pl.ds(0, _SM)],
                out_sem.at[slot, h],
            ).wait()

    w_stripe(0).start()
    dma_in(0, 0)
    for q in range(1, _WSTRIPES):
        w_stripe(q).start()
    dma_in(1, 1)
    dma_in(2, 2)

    w_stripe(0).wait()
    wait_in(0)
    x0 = x_buf[0]
    o_buf[0] = b_ref[...] + jnp.dot(
        x0[:, 0:wk], w_vmem[0:wk, :], preferred_element_type=jnp.float32
    )
    for q in range(1, _WSTRIPES):
        w_stripe(q).wait()
        o_buf[0] += jnp.dot(
            x0[:, q * wk:(q + 1) * wk], w_vmem[q * wk:(q + 1) * wk, :],
            preferred_element_type=jnp.float32,
        )
    dma_out(0, 0)

    def body(step, _):
        cur = jax.lax.rem(step, _NXB)
        pre = jax.lax.rem(step + 2, _NXB)
        ocur = jax.lax.rem(step, _NOB)

        @pl.when(step + 2 < n_steps)
        def _prefetch():
            dma_in(pre, step + 2)

        wait_in(cur)

        @pl.when(step >= _NOB)
        def _drain():
            wait_out(ocur)

        o_buf[ocur] = (
            jnp.dot(x_buf[cur], w_vmem[...], preferred_element_type=jnp.float32)
            + b_ref[...]
        )
        dma_out(ocur, step)
        return ()

    jax.lax.fori_loop(1, n_steps, body, (), unroll=2)
    for t in range(n_steps - _NOB, n_steps):
        wait_out(jax.lax.rem(t, _NOB))


def kernel(x, w, b):
    m, k = x.shape
    n = w.shape[1]
    cost = pl.CostEstimate(
        flops=2 * m * k * n,
        transcendentals=0,
        bytes_accessed=4 * m * k + 4 * k * n + 4 * n + 4 * m * n,
    )
    return pl.pallas_call(
        _linear_kernel,
        out_shape=jax.ShapeDtypeStruct((m, n), jnp.float32),
        in_specs=[
            pl.BlockSpec(memory_space=pltpu.MemorySpace.HBM),
            pl.BlockSpec(memory_space=pltpu.MemorySpace.HBM),
            pl.BlockSpec(memory_space=pltpu.MemorySpace.VMEM),
        ],
        out_specs=pl.BlockSpec(memory_space=pltpu.MemorySpace.HBM),
        scratch_shapes=[
            pltpu.VMEM((k, n), jnp.float32),
            pltpu.VMEM((_NXB, _TM, k), jnp.float32),
            pltpu.VMEM((_NOB, _TM, n), jnp.float32),
            pltpu.SemaphoreType.DMA((_NXB, _S)),
            pltpu.SemaphoreType.DMA((_NOB, _S)),
            pltpu.SemaphoreType.DMA((_WSTRIPES,)),
        ],
        compiler_params=pltpu.CompilerParams(
            vmem_limit_bytes=60 << 20,
        ),
        cost_estimate=cost,
    )(x, w, b)
